# baseline (speedup 1.0000x reference)
import functools

import jax
import jax.numpy as jnp
from jax import lax
from jax.experimental import pallas as pl
from jax.experimental.pallas import tpu as pltpu
from jax.experimental.pallas import tpu_sc as plsc

HID = 768
EPS = 1e-12

_NUM_CORES = 2
_NUM_SUBCORES = 16
_NW = _NUM_CORES * _NUM_SUBCORES

_CHUNK = 64


def _sc_gather(table, idx2d):
    n_chunks = idx2d.shape[0]
    chunks_per_w = n_chunks // _NW
    n_tokens = n_chunks * _CHUNK
    mesh = plsc.VectorSubcoreMesh(core_axis_name="c", subcore_axis_name="s")

    @functools.partial(
        pl.kernel,
        mesh=mesh,
        out_type=jax.ShapeDtypeStruct((n_tokens, HID), jnp.float32),
        scratch_types=[
            pltpu.VMEM((chunks_per_w, _CHUNK), jnp.int32),
            pltpu.VMEM((2, _CHUNK, HID), jnp.float32),
            pltpu.SemaphoreType.DMA,
            pltpu.SemaphoreType.DMA,
        ],
    )
    def gather_kernel(table_hbm, idx_hbm, out_hbm, idx_v, rows_v, gsem, osem):
        wid = lax.axis_index("s") * _NUM_CORES + lax.axis_index("c")
        chunk0 = wid * chunks_per_w
        base = chunk0 * _CHUNK
        pltpu.sync_copy(idx_hbm.at[pl.ds(chunk0, chunks_per_w)], idx_v)

        def start_gather(j):
            return pltpu.async_copy(
                table_hbm.at[idx_v.at[j]], rows_v.at[j % 2], gsem)

        def start_out(j):
            return pltpu.async_copy(
                rows_v.at[j % 2],
                out_hbm.at[pl.ds(base + j * _CHUNK, _CHUNK)], osem)

        outs = [None] * chunks_per_w
        g = start_gather(0)
        for j in range(chunks_per_w):
            g.wait()
            outs[j] = start_out(j)
            if j + 1 < chunks_per_w:
                if j >= 1:
                    outs[j - 1].wait()
                g = start_gather(j + 1)
        if chunks_per_w >= 2:
            outs[chunks_per_w - 2].wait()
        outs[chunks_per_w - 1].wait()

    return gather_kernel(table, idx2d)


def _ln_body(x_ref, pos_ref, type_ref, gamma_ref, beta_ref, o_ref):
    x = x_ref[0] + (pos_ref[...] + type_ref[...])
    mean = jnp.mean(x, axis=-1, keepdims=True)
    msq = jnp.mean(x * x, axis=-1, keepdims=True)
    var = msq - mean * mean
    rstd = lax.rsqrt(var + EPS)
    o_ref[0] = (x - mean) * (rstd * gamma_ref[...]) + beta_ref[...]


def _ln_slab_body(buf_ref, x_ref, pos_ref, type_ref, gamma_ref, beta_ref,
                  o_ref):
    del buf_ref
    _ln_body(x_ref, pos_ref, type_ref, gamma_ref, beta_ref, o_ref)


def _ln_slab(x_slab, pos_emb, type_row, gamma, beta, out_shape, j0,
             buf=None):
    B, s_slab, _ = x_slab.shape
    BS = _LN_BS
    grid = (s_slab // BS, B)
    data_specs = [
        pl.BlockSpec((1, BS, HID), lambda j, b: (b, j, 0)),
        pl.BlockSpec((BS, HID), lambda j, b: (j0 + j, 0)),
        pl.BlockSpec((1, HID), lambda j, b: (0, 0)),
        pl.BlockSpec((1, HID), lambda j, b: (0, 0)),
        pl.BlockSpec((1, HID), lambda j, b: (0, 0)),
    ]
    out_spec = pl.BlockSpec((1, BS, HID), lambda j, b: (b, j0 + j, 0))
    if buf is None:
        return pl.pallas_call(
            _ln_body, grid=grid, in_specs=data_specs, out_specs=out_spec,
            out_shape=out_shape,
        )(x_slab, pos_emb, type_row, gamma, beta)
    return pl.pallas_call(
        _ln_slab_body, grid=grid,
        in_specs=[pl.BlockSpec(memory_space=pl.ANY)] + data_specs,
        out_specs=out_spec, out_shape=out_shape,
        input_output_aliases={0: 0},
    )(buf, x_slab, pos_emb, type_row, gamma, beta)


_K_SLABS = 4
_LN_BS = 1024


def kernel(input_ids, word_emb, pos_emb, type_emb, ln_gamma, ln_beta):
    B, S = input_ids.shape
    ids = input_ids.astype(jnp.int32)
    type_row = type_emb[0:1]
    gamma = ln_gamma.reshape(1, HID)
    beta = ln_beta.reshape(1, HID)
    out_shape = jax.ShapeDtypeStruct((B, S, HID), jnp.float32)
    s_slab = S // _K_SLABS
    gathered = []
    for k in range(_K_SLABS):
        ids_k = lax.dynamic_slice_in_dim(ids, k * s_slab, s_slab, axis=1)
        g = _sc_gather(word_emb,
                       ids_k.reshape(B * s_slab // _CHUNK, _CHUNK))
        gathered.append(g.reshape(B, s_slab, HID))
    buf = None
    for k in range(_K_SLABS):
        buf = _ln_slab(gathered[k], pos_emb, type_row, gamma, beta,
                       out_shape, k * (s_slab // _LN_BS), buf)
    return buf


# speedup vs baseline: 1.0667x; 1.0667x over previous
import functools

import jax
import jax.numpy as jnp
from jax import lax
from jax.experimental import pallas as pl
from jax.experimental.pallas import tpu as pltpu
from jax.experimental.pallas import tpu_sc as plsc

HID = 768
EPS = 1e-12

_NUM_CORES = 2
_NUM_SUBCORES = 16
_NW = _NUM_CORES * _NUM_SUBCORES

_CHUNK = 64


def _sc_gather(table, idx2d):
    n_chunks = idx2d.shape[0]
    chunks_per_w = n_chunks // _NW
    n_tokens = n_chunks * _CHUNK
    mesh = plsc.VectorSubcoreMesh(core_axis_name="c", subcore_axis_name="s")

    @functools.partial(
        pl.kernel,
        mesh=mesh,
        out_type=jax.ShapeDtypeStruct((n_tokens, HID), jnp.float32),
        scratch_types=[
            pltpu.VMEM((chunks_per_w, _CHUNK), jnp.int32),
            pltpu.VMEM((2, _CHUNK, HID), jnp.float32),
            pltpu.SemaphoreType.DMA,
            pltpu.SemaphoreType.DMA,
        ],
    )
    def gather_kernel(table_hbm, idx_hbm, out_hbm, idx_v, rows_v, gsem, osem):
        wid = lax.axis_index("s") * _NUM_CORES + lax.axis_index("c")
        chunk0 = wid * chunks_per_w
        base = chunk0 * _CHUNK
        pltpu.sync_copy(idx_hbm.at[pl.ds(chunk0, chunks_per_w)], idx_v)

        def start_gather(j):
            return pltpu.async_copy(
                table_hbm.at[idx_v.at[j]], rows_v.at[j % 2], gsem)

        def start_out(j):
            return pltpu.async_copy(
                rows_v.at[j % 2],
                out_hbm.at[pl.ds(base + j * _CHUNK, _CHUNK)], osem)

        outs = [None] * chunks_per_w
        g = start_gather(0)
        for j in range(chunks_per_w):
            g.wait()
            outs[j] = start_out(j)
            if j + 1 < chunks_per_w:
                if j >= 1:
                    outs[j - 1].wait()
                g = start_gather(j + 1)
        if chunks_per_w >= 2:
            outs[chunks_per_w - 2].wait()
        outs[chunks_per_w - 1].wait()

    return gather_kernel(table, idx2d)


def _ln_body(x_ref, pos_ref, type_ref, gamma_ref, beta_ref, o_ref):
    x = x_ref[0] + (pos_ref[...] + type_ref[...])
    mean = jnp.mean(x, axis=-1, keepdims=True)
    msq = jnp.mean(x * x, axis=-1, keepdims=True)
    var = msq - mean * mean
    rstd = lax.rsqrt(var + EPS)
    o_ref[0] = (x - mean) * (rstd * gamma_ref[...]) + beta_ref[...]


def _ln_slab_body(buf_ref, x_ref, pos_ref, type_ref, gamma_ref, beta_ref,
                  o_ref):
    del buf_ref
    _ln_body(x_ref, pos_ref, type_ref, gamma_ref, beta_ref, o_ref)


def _ln_slab(x_slab, pos_emb, type_row, gamma, beta, out_shape, j0,
             buf=None):
    B, s_slab, _ = x_slab.shape
    BS = _LN_BS
    grid = (s_slab // BS, B)
    data_specs = [
        pl.BlockSpec((1, BS, HID), lambda j, b: (b, j, 0)),
        pl.BlockSpec((BS, HID), lambda j, b: (j0 + j, 0)),
        pl.BlockSpec((1, HID), lambda j, b: (0, 0)),
        pl.BlockSpec((1, HID), lambda j, b: (0, 0)),
        pl.BlockSpec((1, HID), lambda j, b: (0, 0)),
    ]
    out_spec = pl.BlockSpec((1, BS, HID), lambda j, b: (b, j0 + j, 0))
    if buf is None:
        return pl.pallas_call(
            _ln_body, grid=grid, in_specs=data_specs, out_specs=out_spec,
            out_shape=out_shape,
        )(x_slab, pos_emb, type_row, gamma, beta)
    return pl.pallas_call(
        _ln_slab_body, grid=grid,
        in_specs=[pl.BlockSpec(memory_space=pl.ANY)] + data_specs,
        out_specs=out_spec, out_shape=out_shape,
        input_output_aliases={0: 0},
    )(buf, x_slab, pos_emb, type_row, gamma, beta)


_K_SLABS = 2
_LN_BS = 2048


def kernel(input_ids, word_emb, pos_emb, type_emb, ln_gamma, ln_beta):
    B, S = input_ids.shape
    ids = input_ids.astype(jnp.int32)
    type_row = type_emb[0:1]
    gamma = ln_gamma.reshape(1, HID)
    beta = ln_beta.reshape(1, HID)
    out_shape = jax.ShapeDtypeStruct((B, S, HID), jnp.float32)
    s_slab = S // _K_SLABS
    gathered = []
    for k in range(_K_SLABS):
        ids_k = lax.dynamic_slice_in_dim(ids, k * s_slab, s_slab, axis=1)
        g = _sc_gather(word_emb,
                       ids_k.reshape(B * s_slab // _CHUNK, _CHUNK))
        gathered.append(g.reshape(B, s_slab, HID))
    buf = None
    for k in range(_K_SLABS):
        buf = _ln_slab(gathered[k], pos_emb, type_row, gamma, beta,
                       out_shape, k * (s_slab // _LN_BS), buf)
    return buf
